# baseline (speedup 1.0000x reference)
import jax
import jax.numpy as jnp
from jax.experimental import pallas as pl

N = 100000
E = 1600000
D_IN = 9
HID = 8
HEADS = 8
D = 64
G = 128
EPS = 1e-5


def _gat_conv(x, edge_index, W, a_s, a_d, bias, heads, out_ch):
    src = edge_index[0]
    dst = edge_index[1]
    h = (x @ W).reshape(N, heads, out_ch)
    alpha_src = jnp.sum(h * a_s, axis=-1)
    alpha_dst = jnp.sum(h * a_d, axis=-1)
    e = jax.nn.leaky_relu(alpha_src[src] + alpha_dst[dst], 0.2)
    m = jax.ops.segment_max(e, dst, num_segments=N)
    m = jnp.where(jnp.isfinite(m), m, 0.0)
    ex = jnp.exp(e - m[dst])
    den = jax.ops.segment_sum(ex, dst, num_segments=N)
    alpha = ex / (den[dst] + 1e-16)
    out = jax.ops.segment_sum(h[src] * alpha[:, :, None], dst, num_segments=N)
    return out.reshape(N, heads * out_ch) + bias


def _graph_norm(x, g, b):
    x = x - jnp.mean(x)
    out = x / (jnp.std(x) + EPS)
    return out * g + b


def _identity_kernel(x_ref, o_ref):
    o_ref[...] = x_ref[...]


def kernel(x, edge_index, batch, W1, a_src1, a_dst1, b1, Wfc1, bfc1, Wfc2, bfc2, g1, be1, W2, a_src2, a_dst2, b2, Wfc3, bfc3, Wfc4, bfc4, g2, be2, Wf, bf):
    h = jax.nn.relu(_gat_conv(x, edge_index, W1, a_src1, a_dst1, b1, HEADS, HID))
    hc = h
    h = jax.nn.relu(h @ Wfc1 + bfc1)
    h = jax.nn.relu(h @ Wfc2 + bfc2 + hc)
    h = _graph_norm(h, g1, be1)
    hc = h
    h = _gat_conv(h, edge_index, W2, a_src2, a_dst2, b2, 1, D)
    h = jax.nn.relu(h @ Wfc3 + bfc3)
    h = h @ Wfc4 + bfc4 + hc
    h = _graph_norm(h, g2, be2)
    h = h @ Wf + bf
    s = jax.ops.segment_sum(h, batch, num_segments=G)
    cnt = jax.ops.segment_sum(jnp.ones((N, 1), h.dtype), batch, num_segments=G)
    out = s / jnp.clip(cnt, 1.0, None)
    return pl.pallas_call(
        _identity_kernel,
        out_shape=jax.ShapeDtypeStruct(out.shape, out.dtype),
    )(out)


# speedup vs baseline: 25.6580x; 25.6580x over previous
import functools

import jax
import jax.numpy as jnp
from jax import lax
from jax.experimental import pallas as pl
from jax.experimental.pallas import tpu as pltpu
from jax.experimental.pallas import tpu_sc as plsc

N = 100000
E = 1600000
D_IN = 9
HID = 8
HEADS = 8
D = 64
G = 128
EPS = 1e-5

BN = 512
NB = 196
NPAD = NB * BN

NC, NS = 2, 16
R = 4096
NPASS = 25
TPAD = NPASS * R
RACC = R + 32
PCOLS = 128
EPW = E // NS
SB = 2000
NSB = EPW // SB
GB = 128
CAP = 10496
THR = 8192
ZROWS = 43


def _sc_msgpass(mode):
    adw = 8 if mode == 1 else 1
    mesh = plsc.VectorSubcoreMesh(
        core_axis_name="c", subcore_axis_name="s",
        num_cores=NC, num_subcores=NS)

    @functools.partial(
        pl.kernel,
        out_type=jax.ShapeDtypeStruct((TPAD, PCOLS), jnp.float32),
        mesh=mesh,
        scratch_types=[
            pltpu.VMEM_SHARED((RACC, PCOLS), jnp.float32),
            pltpu.VMEM((CAP,), jnp.int32),
            pltpu.VMEM((CAP,), jnp.int32),
            pltpu.VMEM((SB,), jnp.int32),
            pltpu.VMEM((SB,), jnp.int32),
            pltpu.VMEM((GB, PCOLS), jnp.float32),
            pltpu.VMEM((R * adw,), jnp.float32),
            pltpu.VMEM((GB,), jnp.int32),
            pltpu.VMEM((16,), jnp.float32),
            pltpu.VMEM((ZROWS, PCOLS), jnp.float32),
        ],
        compiler_params=pltpu.CompilerParams(needs_layout_passes=False),
    )
    def k(ei, p_t, ad_t, out, acc, wl_s, wl_d, st_s, st_d,
          rows, ad_sl, lidx, wsc, zbuf):
        c = lax.axis_index("c")
        s = lax.axis_index("s")
        zvec = jnp.zeros((16,), jnp.float32)

        @pl.loop(0, ZROWS)
        def _(r):
            for kk in range(PCOLS // 16):
                zbuf[r, pl.ds(kk * 16, 16)] = zvec

        iota = lax.iota(jnp.int32, 16)

        def edge_body(e, _):
            loc_b = plsc.load_gather(lidx, [iota * 0 + e])
            if mode == 1:
                adidx = loc_b * 8 + (iota & 7)
            else:
                adidx = loc_b
            ad = plsc.load_gather(ad_sl, [adidx])
            asad = rows[e, pl.ds(64, 16)]
            t = asad + ad
            t = jnp.maximum(t, 0.2 * t)
            w = jnp.exp(t)
            if mode == 1:
                wsc[...] = w
                for kk in range(4):
                    bidx = lax.shift_right_logical(iota, 3) + (2 * kk)
                    wb = plsc.load_gather(wsc, [bidx])
                    rows[e, pl.ds(kk * 16, 16)] = wb * rows[e, pl.ds(kk * 16, 16)]
                rows[e, pl.ds(64, 16)] = w
            else:
                for kk in range(4):
                    rows[e, pl.ds(kk * 16, 16)] = w * rows[e, pl.ds(kk * 16, 16)]
                rows[e, pl.ds(64, 16)] = w
            return 0

        def make_drain(lo):
            def drain(ptr):
                for kk in range(16):
                    wl_s[pl.ds(ptr + 16 * kk, 16)] = iota * 0
                    wl_d[pl.ds(ptr + 16 * kk, 16)] = iota * 0 + (lo + R)
                nb = lax.shift_right_logical(ptr + (GB - 1), 7)

                def blk(g, _):
                    isl = pl.ds(g * GB, GB)

                    def lidx_body(i, _):
                        d16 = wl_d[pl.ds(g * GB + i * 16, 16)]
                        lidx[pl.ds(i * 16, 16)] = d16 - lo
                        return 0

                    lax.fori_loop(0, GB // 16, lidx_body, 0)
                    pltpu.sync_copy(p_t.at[wl_s.at[isl]], rows)
                    lax.fori_loop(0, GB, edge_body, 0)
                    pltpu.sync_copy(rows, acc.at[lidx], add=True)
                    return 0

                lax.fori_loop(0, nb, blk, 0)
                return jnp.int32(0)
            return drain

        for j in range(13):
            p = c + 2 * j

            @pl.when(p < NPASS)
            def _():
                lo = p * R
                hi = lo + R
                drain = make_drain(lo)

                pltpu.sync_copy(ad_t.at[pl.ds(lo * adw, R * adw)], ad_sl)
                for z in range(6):
                    pltpu.sync_copy(
                        zbuf, acc.at[pl.ds(s * (6 * ZROWS) + z * ZROWS, ZROWS)])
                plsc.subcore_barrier()

                def scan_blk(b, ptr):
                    base_e = s * EPW + b * SB
                    pltpu.sync_copy(ei.at[pl.ds(base_e, SB)], st_s)
                    pltpu.sync_copy(ei.at[pl.ds(E + base_e, SB)], st_d)

                    def lane(i, ptr):
                        d16 = st_d[pl.ds(i * 16, 16)]
                        m = (d16 >= lo) & (d16 < hi)
                        plsc.store_compressed(wl_d.at[pl.ds(ptr, 16)], d16, mask=m)
                        s16 = st_s[pl.ds(i * 16, 16)]
                        plsc.store_compressed(wl_s.at[pl.ds(ptr, 16)], s16, mask=m)
                        cnt = jnp.max(plsc.all_reduce_population_count(m))
                        return ptr + cnt

                    ptr = lax.fori_loop(0, SB // 16, lane, ptr)
                    ptr = lax.cond(ptr >= THR, drain, lambda q: q, ptr)
                    return ptr

                ptr = lax.fori_loop(0, NSB, scan_blk, jnp.int32(0))
                drain(ptr)
                plsc.subcore_barrier()
                pltpu.sync_copy(acc.at[pl.ds(s * (R // NS), R // NS)],
                                out.at[pl.ds(lo + s * (R // NS), R // NS)])
                plsc.subcore_barrier()

    return k


@functools.lru_cache(maxsize=2)
def _sc_layer(mode):
    return _sc_msgpass(mode)


def _sc_layer1(ei, p_t, ad_2d):
    ad = jnp.pad(ad_2d, ((0, TPAD - NPAD), (0, 0))).reshape(TPAD * 8)
    return _sc_layer(1)(ei.reshape(2 * E), p_t, ad)


def _sc_layer2(ei, p_t, ad_2d):
    ad = jnp.pad(ad_2d, ((0, TPAD - NPAD), (0, 0))).reshape(TPAD)
    return _sc_layer(2)(ei.reshape(2 * E), p_t, ad)


def _k1_body(x_ref, w1_ref, a1_ref, p_ref, ad_ref):
    h = jnp.dot(x_ref[...], w1_ref[...], preferred_element_type=jnp.float32)
    t = jnp.dot(h, a1_ref[...], preferred_element_type=jnp.float32)
    zp = jnp.zeros((BN, PCOLS - 72), jnp.float32)
    p_ref[...] = jnp.concatenate([h, t[:, :8], zp], axis=1)
    ad_ref[...] = t[:, 8:]


def _k3_body(acc_ref, b1_ref, rep_ref, wfc1_ref, bfc1_ref, wfc2_ref, bfc2_ref,
             u_ref, st_ref):
    i = pl.program_id(0)
    acc = acc_ref[...]
    den = jnp.dot(acc[:, 64:72], rep_ref[...],
                  preferred_element_type=jnp.float32)
    h = jax.nn.relu(acc[:, :64] / (den + 1e-16) + b1_ref[...])
    t = jax.nn.relu(jnp.dot(h, wfc1_ref[...],
                            preferred_element_type=jnp.float32) + bfc1_ref[...])
    u = jax.nn.relu(jnp.dot(t, wfc2_ref[...],
                            preferred_element_type=jnp.float32)
                    + bfc2_ref[...] + h)
    u_ref[...] = u
    rid = lax.broadcasted_iota(jnp.int32, (BN, 1), 0) + i * BN
    um = jnp.where(rid < N, u, 0.0)

    @pl.when(i == 0)
    def _():
        st_ref[...] = jnp.zeros_like(st_ref)

    st_ref[...] += jnp.concatenate(
        [jnp.sum(um).reshape(1, 1), jnp.sum(um * um).reshape(1, 1)], axis=1)


def _k4_body(u_ref, st_ref, g1_ref, be1_ref, w2_ref, a2_ref,
             gn_ref, p_ref, ad_ref):
    mean, std = _stats(st_ref)
    gn = (u_ref[...] - mean) / (std + EPS) * g1_ref[...] + be1_ref[...]
    gn_ref[...] = gn
    h2 = jnp.dot(gn, w2_ref[...], preferred_element_type=jnp.float32)
    t2 = jnp.dot(h2, a2_ref[...], preferred_element_type=jnp.float32)
    zp = jnp.zeros((BN, PCOLS - 80), jnp.float32)
    as2 = jnp.broadcast_to(t2[:, 0:1], (BN, 16))
    p_ref[...] = jnp.concatenate([h2, as2, zp], axis=1)
    ad_ref[...] = t2[:, 1:2]


def _k5_body(acc_ref, b2_ref, gn_ref, wfc3_ref, bfc3_ref, wfc4_ref, bfc4_ref,
             v_ref, st_ref):
    i = pl.program_id(0)
    acc = acc_ref[...]
    h = acc[:, :64] / (acc[:, 64:65] + 1e-16) + b2_ref[...]
    t = jax.nn.relu(jnp.dot(h, wfc3_ref[...],
                            preferred_element_type=jnp.float32) + bfc3_ref[...])
    v = (jnp.dot(t, wfc4_ref[...], preferred_element_type=jnp.float32)
         + bfc4_ref[...] + gn_ref[...])
    v_ref[...] = v
    rid = lax.broadcasted_iota(jnp.int32, (BN, 1), 0) + i * BN
    vm = jnp.where(rid < N, v, 0.0)

    @pl.when(i == 0)
    def _():
        st_ref[...] = jnp.zeros_like(st_ref)

    st_ref[...] += jnp.concatenate(
        [jnp.sum(vm).reshape(1, 1), jnp.sum(vm * vm).reshape(1, 1)], axis=1)


def _stats(st_ref):
    st = st_ref[...]
    mean = st[0, 0] / (N * D)
    var = st[0, 1] / (N * D) - mean * mean
    std = jnp.sqrt(jnp.maximum(var, 0.0))
    return mean, std


def _k6_body(v_ref, st_ref, g2_ref, be2_ref, wf_ref, bf_ref, b_ref, out_ref):
    i = pl.program_id(0)
    mean, std = _stats(st_ref)
    w = (v_ref[...] - mean) / (std + EPS) * g2_ref[...] + be2_ref[...]
    y = jnp.dot(w, wf_ref[...], preferred_element_type=jnp.float32) + bf_ref[...]
    bvals = b_ref[...].reshape(1, BN)
    gids = lax.broadcasted_iota(jnp.int32, (G, 1), 0).astype(jnp.float32)
    oh = (bvals == gids).astype(jnp.float32)
    ones = jnp.ones((BN, 1), jnp.float32)
    cy = jnp.dot(oh, y, preferred_element_type=jnp.float32)
    cc = jnp.dot(oh, ones, preferred_element_type=jnp.float32)

    @pl.when(i == 0)
    def _():
        out_ref[...] = jnp.zeros_like(out_ref)

    out_ref[...] += jnp.concatenate([cy, cc], axis=1)


def _k7_body(p_ref, o_ref):
    o_ref[...] = p_ref[:, 0:1] / jnp.clip(p_ref[:, 1:2], 1.0, None)


def _full(shape):
    return pl.BlockSpec(shape, lambda i: tuple(0 for _ in shape))


def kernel(x, edge_index, batch, W1, a_src1, a_dst1, b1, Wfc1, bfc1, Wfc2,
           bfc2, g1, be1, W2, a_src2, a_dst2, b2, Wfc3, bfc3, Wfc4, bfc4,
           g2, be2, Wf, bf):
    f32 = jnp.float32
    as1m = a_src1.reshape(HEADS, HID)
    ad1m = a_dst1.reshape(HEADS, HID)
    eye = jnp.eye(HEADS, dtype=f32)
    A_s = (eye[:, None, :] * as1m[:, :, None]).reshape(HEADS * HID, HEADS)
    A_d = (eye[:, None, :] * ad1m[:, :, None]).reshape(HEADS * HID, HEADS)
    A1 = jnp.concatenate([A_s, A_d], axis=1)
    A2 = jnp.concatenate([a_src2.reshape(D, 1), a_dst2.reshape(D, 1)], axis=1)
    Rep = jnp.broadcast_to(eye[:, :, None], (8, 8, 8)).reshape(8, 64)
    xp = jnp.pad(x, ((0, NPAD - N), (0, 0)))
    batchf = jnp.pad(batch, (0, NPAD - N), constant_values=G).astype(f32)
    batchf = batchf.reshape(NB, 1, BN)
    b1r, b2r = b1.reshape(1, D), b2.reshape(1, D)
    bfc1r, bfc2r = bfc1.reshape(1, D), bfc2.reshape(1, D)
    bfc3r, bfc4r = bfc3.reshape(1, D), bfc4.reshape(1, D)
    g1r, be1r = g1.reshape(1, D), be1.reshape(1, D)
    g2r, be2r = g2.reshape(1, D), be2.reshape(1, D)
    bfr = bf.reshape(1, 1)

    def rs(w):
        return pl.BlockSpec((BN, w), lambda i: (i, 0))

    P1, AD1 = pl.pallas_call(
        _k1_body,
        grid=(NB,),
        in_specs=[rs(D_IN), _full((D_IN, D)), _full((D, 16))],
        out_specs=[rs(PCOLS), rs(8)],
        out_shape=[jax.ShapeDtypeStruct((NPAD, PCOLS), f32),
                   jax.ShapeDtypeStruct((NPAD, 8), f32)],
    )(xp, W1, A1)

    ACC1 = _sc_layer1(edge_index, P1, AD1)

    u, st1 = pl.pallas_call(
        _k3_body,
        grid=(NB,),
        in_specs=[rs(PCOLS), _full((1, D)), _full((8, D)), _full((D, D)),
                  _full((1, D)), _full((D, D)), _full((1, D))],
        out_specs=[rs(D), _full((1, 2))],
        out_shape=[jax.ShapeDtypeStruct((NPAD, D), f32),
                   jax.ShapeDtypeStruct((1, 2), f32)],
    )(ACC1, b1r, Rep, Wfc1, bfc1r, Wfc2, bfc2r)

    gn, P2, AD2 = pl.pallas_call(
        _k4_body,
        grid=(NB,),
        in_specs=[rs(D), _full((1, 2)), _full((1, D)), _full((1, D)),
                  _full((D, D)), _full((D, 2))],
        out_specs=[rs(D), rs(PCOLS), rs(1)],
        out_shape=[jax.ShapeDtypeStruct((NPAD, D), f32),
                   jax.ShapeDtypeStruct((NPAD, PCOLS), f32),
                   jax.ShapeDtypeStruct((NPAD, 1), f32)],
    )(u, st1, g1r, be1r, W2, A2)

    ACC2 = _sc_layer2(edge_index, P2, AD2)

    v, st2 = pl.pallas_call(
        _k5_body,
        grid=(NB,),
        in_specs=[rs(PCOLS), _full((1, D)), rs(D), _full((D, D)),
                  _full((1, D)), _full((D, D)), _full((1, D))],
        out_specs=[rs(D), _full((1, 2))],
        out_shape=[jax.ShapeDtypeStruct((NPAD, D), f32),
                   jax.ShapeDtypeStruct((1, 2), f32)],
    )(ACC2, b2r, gn, Wfc3, bfc3r, Wfc4, bfc4r)

    pool = pl.pallas_call(
        _k6_body,
        grid=(NB,),
        in_specs=[rs(D), _full((1, 2)), _full((1, D)), _full((1, D)),
                  _full((D, 1)), _full((1, 1)),
                  pl.BlockSpec((1, 1, BN), lambda i: (i, 0, 0))],
        out_specs=_full((G, 2)),
        out_shape=jax.ShapeDtypeStruct((G, 2), f32),
    )(v, st2, g2r, be2r, Wf, bfr, batchf)

    out = pl.pallas_call(
        _k7_body,
        out_shape=jax.ShapeDtypeStruct((G, 1), f32),
    )(pool)
    return out


# speedup vs baseline: 25.9896x; 1.0129x over previous
import functools

import jax
import jax.numpy as jnp
from jax import lax
from jax.experimental import pallas as pl
from jax.experimental.pallas import tpu as pltpu
from jax.experimental.pallas import tpu_sc as plsc

N = 100000
E = 1600000
D_IN = 9
HID = 8
HEADS = 8
D = 64
G = 128
EPS = 1e-5

BN = 512
NB = 196
NPAD = NB * BN

NC, NS = 2, 16
R = 4096
NPASS = 25
TPAD = NPASS * R
RACC = R + 32
PCOLS = 128
EPW = E // NS
SB = 2000
NSB = EPW // SB
GB = 128
CAP = 6912
THR = 4096
ZROWS = 43


def _sc_msgpass(mode):
    adw = 8 if mode == 1 else 1
    mesh = plsc.VectorSubcoreMesh(
        core_axis_name="c", subcore_axis_name="s",
        num_cores=NC, num_subcores=NS)

    @functools.partial(
        pl.kernel,
        out_type=jax.ShapeDtypeStruct((TPAD, PCOLS), jnp.float32),
        mesh=mesh,
        scratch_types=[
            pltpu.VMEM_SHARED((RACC, PCOLS), jnp.float32),
            pltpu.VMEM((CAP,), jnp.int32),
            pltpu.VMEM((CAP,), jnp.int32),
            pltpu.VMEM((SB,), jnp.int32),
            pltpu.VMEM((SB,), jnp.int32),
            pltpu.VMEM((SB,), jnp.int32),
            pltpu.VMEM((SB,), jnp.int32),
            pltpu.VMEM((GB, PCOLS), jnp.float32),
            pltpu.VMEM((GB, PCOLS), jnp.float32),
            pltpu.VMEM((R * adw,), jnp.float32),
            pltpu.VMEM((GB,), jnp.int32),
            pltpu.VMEM((GB,), jnp.int32),
            pltpu.VMEM((16,), jnp.float32),
            pltpu.VMEM((ZROWS, PCOLS), jnp.float32),
            pltpu.SemaphoreType.DMA,
            pltpu.SemaphoreType.DMA,
            pltpu.SemaphoreType.DMA,
            pltpu.SemaphoreType.DMA,
            pltpu.SemaphoreType.DMA,
            pltpu.SemaphoreType.DMA,
        ],
        compiler_params=pltpu.CompilerParams(needs_layout_passes=False),
    )
    def k(ei, p_t, ad_t, out, acc, wl_s, wl_d, st_s0, st_d0, st_s1, st_d1,
          rows0, rows1, ad_sl, lidx0, lidx1, wsc, zbuf,
          sem_a, sem_b, gsem0, gsem1, ssem0, ssem1):
        c = lax.axis_index("c")
        s = lax.axis_index("s")
        zvec = jnp.zeros((16,), jnp.float32)

        @pl.loop(0, ZROWS)
        def _(r):
            for kk in range(PCOLS // 16):
                zbuf[r, pl.ds(kk * 16, 16)] = zvec

        iota = lax.iota(jnp.int32, 16)

        def make_edge_body(rows, lidx):
            def edge_body(e, _):
                loc_b = plsc.load_gather(lidx, [iota * 0 + e])
                if mode == 1:
                    adidx = loc_b * 8 + (iota & 7)
                else:
                    adidx = loc_b
                ad = plsc.load_gather(ad_sl, [adidx])
                asad = rows[e, pl.ds(64, 16)]
                t = asad + ad
                t = jnp.maximum(t, 0.2 * t)
                w = jnp.exp(t)
                if mode == 1:
                    wsc[...] = w
                    for kk in range(4):
                        bidx = lax.shift_right_logical(iota, 3) + (2 * kk)
                        wb = plsc.load_gather(wsc, [bidx])
                        rows[e, pl.ds(kk * 16, 16)] = (
                            wb * rows[e, pl.ds(kk * 16, 16)])
                    rows[e, pl.ds(64, 16)] = w
                else:
                    for kk in range(4):
                        rows[e, pl.ds(kk * 16, 16)] = (
                            w * rows[e, pl.ds(kk * 16, 16)])
                    rows[e, pl.ds(64, 16)] = w
                return 0
            return edge_body

        def make_drain(lo):
            def compute_blk(g, rows, lidx):
                def lidx_body(i, _):
                    d16 = wl_d[pl.ds(g * GB + i * 16, 16)]
                    lidx[pl.ds(i * 16, 16)] = d16 - lo
                    return 0

                lax.fori_loop(0, GB // 16, lidx_body, 0)
                lax.fori_loop(0, GB, make_edge_body(rows, lidx), 0)

            def drain(ptr):
                for kk in range(16):
                    wl_s[pl.ds(ptr + 16 * kk, 16)] = iota * 0
                    wl_d[pl.ds(ptr + 16 * kk, 16)] = iota * 0 + (lo + R)
                nb2 = lax.shift_right_logical(ptr + (2 * GB - 1), 8)

                def pair(ii, _):
                    g0 = 2 * ii
                    g1 = 2 * ii + 1
                    cg0 = pltpu.async_copy(
                        p_t.at[wl_s.at[pl.ds(g0 * GB, GB)]], rows0, gsem0)
                    cg1 = pltpu.async_copy(
                        p_t.at[wl_s.at[pl.ds(g1 * GB, GB)]], rows1, gsem1)
                    cg0.wait()
                    compute_blk(g0, rows0, lidx0)
                    cs0 = pltpu.async_copy(rows0, acc.at[lidx0], ssem0,
                                           add=True)
                    cg1.wait()
                    compute_blk(g1, rows1, lidx1)
                    cs1 = pltpu.async_copy(rows1, acc.at[lidx1], ssem1,
                                           add=True)
                    cs0.wait()
                    cs1.wait()
                    return 0

                lax.fori_loop(0, nb2, pair, 0)
                return jnp.int32(0)
            return drain

        npass_c = jnp.where(c == 0, (NPASS + 1) // 2, NPASS // 2)

        def pass_body(j, _):
            p = c + 2 * j
            if True:
                lo = p * R
                hi = lo + R
                drain = make_drain(lo)

                pltpu.sync_copy(ad_t.at[pl.ds(lo * adw, R * adw)], ad_sl)
                for z in range(6):
                    pltpu.sync_copy(
                        zbuf, acc.at[pl.ds(s * (6 * ZROWS) + z * ZROWS, ZROWS)])
                plsc.subcore_barrier()

                def fire(b, ss, sd, sem):
                    base_e = s * EPW + b * SB
                    pltpu.async_copy(ei.at[pl.ds(base_e, SB)], ss, sem)
                    pltpu.async_copy(ei.at[pl.ds(E + base_e, SB)], sd, sem)

                def wait_scan(ss, sd, sem):
                    pltpu.make_async_copy(ei.at[pl.ds(0, SB)], ss, sem).wait()
                    pltpu.make_async_copy(ei.at[pl.ds(0, SB)], sd, sem).wait()

                def process(ss, sd, ptr):
                    def lane(i, ptr):
                        d16 = sd[pl.ds(i * 16, 16)]
                        m = (d16 >= lo) & (d16 < hi)
                        plsc.store_compressed(wl_d.at[pl.ds(ptr, 16)], d16,
                                              mask=m)
                        s16 = ss[pl.ds(i * 16, 16)]
                        plsc.store_compressed(wl_s.at[pl.ds(ptr, 16)], s16,
                                              mask=m)
                        cnt = jnp.max(plsc.all_reduce_population_count(m))
                        return ptr + cnt

                    return lax.fori_loop(0, SB // 16, lane, ptr)

                fire(0, st_s0, st_d0, sem_a)

                def scan_pair(i, ptr):
                    fire(2 * i + 1, st_s1, st_d1, sem_b)
                    wait_scan(st_s0, st_d0, sem_a)
                    ptr = process(st_s0, st_d0, ptr)

                    @pl.when(i < NSB // 2 - 1)
                    def _():
                        fire(2 * i + 2, st_s0, st_d0, sem_a)

                    wait_scan(st_s1, st_d1, sem_b)
                    ptr = process(st_s1, st_d1, ptr)
                    ptr = lax.cond(ptr >= THR, drain, lambda q: q, ptr)
                    return ptr

                ptr = lax.fori_loop(0, NSB // 2, scan_pair, jnp.int32(0))
                drain(ptr)
                plsc.subcore_barrier()
                pltpu.sync_copy(acc.at[pl.ds(s * (R // NS), R // NS)],
                                out.at[pl.ds(lo + s * (R // NS), R // NS)])
                plsc.subcore_barrier()
            return 0

        lax.fori_loop(0, npass_c, pass_body, 0)

    return k


@functools.lru_cache(maxsize=2)
def _sc_layer(mode):
    return _sc_msgpass(mode)


def _sc_layer1(ei, p_t, ad_2d):
    ad = jnp.pad(ad_2d, ((0, TPAD - NPAD), (0, 0))).reshape(TPAD * 8)
    return _sc_layer(1)(ei.reshape(2 * E), p_t, ad)


def _sc_layer2(ei, p_t, ad_2d):
    ad = jnp.pad(ad_2d, ((0, TPAD - NPAD), (0, 0))).reshape(TPAD)
    return _sc_layer(2)(ei.reshape(2 * E), p_t, ad)


def _k1_body(x_ref, w1_ref, a1_ref, p_ref, ad_ref):
    h = jnp.dot(x_ref[...], w1_ref[...], preferred_element_type=jnp.float32)
    t = jnp.dot(h, a1_ref[...], preferred_element_type=jnp.float32)
    zp = jnp.zeros((BN, PCOLS - 72), jnp.float32)
    p_ref[...] = jnp.concatenate([h, t[:, :8], zp], axis=1)
    ad_ref[...] = t[:, 8:]


def _k3_body(acc_ref, b1_ref, rep_ref, wfc1_ref, bfc1_ref, wfc2_ref, bfc2_ref,
             u_ref, st_ref):
    i = pl.program_id(0)
    acc = acc_ref[...]
    den = jnp.dot(acc[:, 64:72], rep_ref[...],
                  preferred_element_type=jnp.float32)
    h = jax.nn.relu(acc[:, :64] / (den + 1e-16) + b1_ref[...])
    t = jax.nn.relu(jnp.dot(h, wfc1_ref[...],
                            preferred_element_type=jnp.float32) + bfc1_ref[...])
    u = jax.nn.relu(jnp.dot(t, wfc2_ref[...],
                            preferred_element_type=jnp.float32)
                    + bfc2_ref[...] + h)
    u_ref[...] = u
    rid = lax.broadcasted_iota(jnp.int32, (BN, 1), 0) + i * BN
    um = jnp.where(rid < N, u, 0.0)

    @pl.when(i == 0)
    def _():
        st_ref[...] = jnp.zeros_like(st_ref)

    st_ref[...] += jnp.concatenate(
        [jnp.sum(um).reshape(1, 1), jnp.sum(um * um).reshape(1, 1)], axis=1)


def _k4_body(u_ref, st_ref, g1_ref, be1_ref, w2_ref, a2_ref,
             gn_ref, p_ref, ad_ref):
    mean, std = _stats(st_ref)
    gn = (u_ref[...] - mean) / (std + EPS) * g1_ref[...] + be1_ref[...]
    gn_ref[...] = gn
    h2 = jnp.dot(gn, w2_ref[...], preferred_element_type=jnp.float32)
    t2 = jnp.dot(h2, a2_ref[...], preferred_element_type=jnp.float32)
    zp = jnp.zeros((BN, PCOLS - 80), jnp.float32)
    as2 = jnp.broadcast_to(t2[:, 0:1], (BN, 16))
    p_ref[...] = jnp.concatenate([h2, as2, zp], axis=1)
    ad_ref[...] = t2[:, 1:2]


def _k5_body(acc_ref, b2_ref, gn_ref, wfc3_ref, bfc3_ref, wfc4_ref, bfc4_ref,
             v_ref, st_ref):
    i = pl.program_id(0)
    acc = acc_ref[...]
    h = acc[:, :64] / (acc[:, 64:65] + 1e-16) + b2_ref[...]
    t = jax.nn.relu(jnp.dot(h, wfc3_ref[...],
                            preferred_element_type=jnp.float32) + bfc3_ref[...])
    v = (jnp.dot(t, wfc4_ref[...], preferred_element_type=jnp.float32)
         + bfc4_ref[...] + gn_ref[...])
    v_ref[...] = v
    rid = lax.broadcasted_iota(jnp.int32, (BN, 1), 0) + i * BN
    vm = jnp.where(rid < N, v, 0.0)

    @pl.when(i == 0)
    def _():
        st_ref[...] = jnp.zeros_like(st_ref)

    st_ref[...] += jnp.concatenate(
        [jnp.sum(vm).reshape(1, 1), jnp.sum(vm * vm).reshape(1, 1)], axis=1)


def _stats(st_ref):
    st = st_ref[...]
    mean = st[0, 0] / (N * D)
    var = st[0, 1] / (N * D) - mean * mean
    std = jnp.sqrt(jnp.maximum(var, 0.0))
    return mean, std


def _k6_body(v_ref, st_ref, g2_ref, be2_ref, wf_ref, bf_ref, b_ref, out_ref):
    i = pl.program_id(0)
    mean, std = _stats(st_ref)
    w = (v_ref[...] - mean) / (std + EPS) * g2_ref[...] + be2_ref[...]
    y = jnp.dot(w, wf_ref[...], preferred_element_type=jnp.float32) + bf_ref[...]
    bvals = b_ref[...].reshape(1, BN)
    gids = lax.broadcasted_iota(jnp.int32, (G, 1), 0).astype(jnp.float32)
    oh = (bvals == gids).astype(jnp.float32)
    ones = jnp.ones((BN, 1), jnp.float32)
    cy = jnp.dot(oh, y, preferred_element_type=jnp.float32)
    cc = jnp.dot(oh, ones, preferred_element_type=jnp.float32)

    @pl.when(i == 0)
    def _():
        out_ref[...] = jnp.zeros_like(out_ref)

    out_ref[...] += jnp.concatenate([cy, cc], axis=1)


def _k7_body(p_ref, o_ref):
    o_ref[...] = p_ref[:, 0:1] / jnp.clip(p_ref[:, 1:2], 1.0, None)


def _full(shape):
    return pl.BlockSpec(shape, lambda i: tuple(0 for _ in shape))


def kernel(x, edge_index, batch, W1, a_src1, a_dst1, b1, Wfc1, bfc1, Wfc2,
           bfc2, g1, be1, W2, a_src2, a_dst2, b2, Wfc3, bfc3, Wfc4, bfc4,
           g2, be2, Wf, bf):
    f32 = jnp.float32
    as1m = a_src1.reshape(HEADS, HID)
    ad1m = a_dst1.reshape(HEADS, HID)
    eye = jnp.eye(HEADS, dtype=f32)
    A_s = (eye[:, None, :] * as1m[:, :, None]).reshape(HEADS * HID, HEADS)
    A_d = (eye[:, None, :] * ad1m[:, :, None]).reshape(HEADS * HID, HEADS)
    A1 = jnp.concatenate([A_s, A_d], axis=1)
    A2 = jnp.concatenate([a_src2.reshape(D, 1), a_dst2.reshape(D, 1)], axis=1)
    Rep = jnp.broadcast_to(eye[:, :, None], (8, 8, 8)).reshape(8, 64)
    xp = jnp.pad(x, ((0, NPAD - N), (0, 0)))
    batchf = jnp.pad(batch, (0, NPAD - N), constant_values=G).astype(f32)
    batchf = batchf.reshape(NB, 1, BN)
    b1r, b2r = b1.reshape(1, D), b2.reshape(1, D)
    bfc1r, bfc2r = bfc1.reshape(1, D), bfc2.reshape(1, D)
    bfc3r, bfc4r = bfc3.reshape(1, D), bfc4.reshape(1, D)
    g1r, be1r = g1.reshape(1, D), be1.reshape(1, D)
    g2r, be2r = g2.reshape(1, D), be2.reshape(1, D)
    bfr = bf.reshape(1, 1)

    def rs(w):
        return pl.BlockSpec((BN, w), lambda i: (i, 0))

    P1, AD1 = pl.pallas_call(
        _k1_body,
        grid=(NB,),
        in_specs=[rs(D_IN), _full((D_IN, D)), _full((D, 16))],
        out_specs=[rs(PCOLS), rs(8)],
        out_shape=[jax.ShapeDtypeStruct((NPAD, PCOLS), f32),
                   jax.ShapeDtypeStruct((NPAD, 8), f32)],
    )(xp, W1, A1)

    ACC1 = _sc_layer1(edge_index, P1, AD1)

    u, st1 = pl.pallas_call(
        _k3_body,
        grid=(NB,),
        in_specs=[rs(PCOLS), _full((1, D)), _full((8, D)), _full((D, D)),
                  _full((1, D)), _full((D, D)), _full((1, D))],
        out_specs=[rs(D), _full((1, 2))],
        out_shape=[jax.ShapeDtypeStruct((NPAD, D), f32),
                   jax.ShapeDtypeStruct((1, 2), f32)],
    )(ACC1, b1r, Rep, Wfc1, bfc1r, Wfc2, bfc2r)

    gn, P2, AD2 = pl.pallas_call(
        _k4_body,
        grid=(NB,),
        in_specs=[rs(D), _full((1, 2)), _full((1, D)), _full((1, D)),
                  _full((D, D)), _full((D, 2))],
        out_specs=[rs(D), rs(PCOLS), rs(1)],
        out_shape=[jax.ShapeDtypeStruct((NPAD, D), f32),
                   jax.ShapeDtypeStruct((NPAD, PCOLS), f32),
                   jax.ShapeDtypeStruct((NPAD, 1), f32)],
    )(u, st1, g1r, be1r, W2, A2)

    ACC2 = _sc_layer2(edge_index, P2, AD2)

    v, st2 = pl.pallas_call(
        _k5_body,
        grid=(NB,),
        in_specs=[rs(PCOLS), _full((1, D)), rs(D), _full((D, D)),
                  _full((1, D)), _full((D, D)), _full((1, D))],
        out_specs=[rs(D), _full((1, 2))],
        out_shape=[jax.ShapeDtypeStruct((NPAD, D), f32),
                   jax.ShapeDtypeStruct((1, 2), f32)],
    )(ACC2, b2r, gn, Wfc3, bfc3r, Wfc4, bfc4r)

    pool = pl.pallas_call(
        _k6_body,
        grid=(NB,),
        in_specs=[rs(D), _full((1, 2)), _full((1, D)), _full((1, D)),
                  _full((D, 1)), _full((1, 1)),
                  pl.BlockSpec((1, 1, BN), lambda i: (i, 0, 0))],
        out_specs=_full((G, 2)),
        out_shape=jax.ShapeDtypeStruct((G, 2), f32),
    )(v, st2, g2r, be2r, Wf, bfr, batchf)

    out = pl.pallas_call(
        _k7_body,
        out_shape=jax.ShapeDtypeStruct((G, 1), f32),
    )(pool)
    return out


# speedup vs baseline: 26.0934x; 1.0040x over previous
import functools

import jax
import jax.numpy as jnp
from jax import lax
from jax.experimental import pallas as pl
from jax.experimental.pallas import tpu as pltpu
from jax.experimental.pallas import tpu_sc as plsc

N = 100000
E = 1600000
D_IN = 9
HID = 8
HEADS = 8
D = 64
G = 128
EPS = 1e-5

BN = 512
NB = 196
NPAD = NB * BN

NC, NS = 2, 16
R = 4096
NPASS = 25
TPAD = NPASS * R
RACC = R + 32
PCOLS = 128
EPW = E // NS
SB = 2000
NSB = EPW // SB
GB = 128
CAP = 6912
THR = 4096
ZROWS = 43


def _sc_msgpass(mode):
    adw = 8 if mode == 1 else 1
    mesh = plsc.VectorSubcoreMesh(
        core_axis_name="c", subcore_axis_name="s",
        num_cores=NC, num_subcores=NS)

    @functools.partial(
        pl.kernel,
        out_type=jax.ShapeDtypeStruct((TPAD, PCOLS), jnp.float32),
        mesh=mesh,
        scratch_types=[
            pltpu.VMEM_SHARED((RACC, PCOLS), jnp.float32),
            pltpu.VMEM((CAP,), jnp.int32),
            pltpu.VMEM((CAP,), jnp.int32),
            pltpu.VMEM((SB,), jnp.int32),
            pltpu.VMEM((SB,), jnp.int32),
            pltpu.VMEM((SB,), jnp.int32),
            pltpu.VMEM((SB,), jnp.int32),
            pltpu.VMEM((GB, PCOLS), jnp.float32),
            pltpu.VMEM((GB, PCOLS), jnp.float32),
            pltpu.VMEM((R * adw,), jnp.float32),
            pltpu.VMEM((GB,), jnp.int32),
            pltpu.VMEM((GB,), jnp.int32),
            pltpu.VMEM((32,), jnp.float32),
            pltpu.VMEM((ZROWS, PCOLS), jnp.float32),
            pltpu.SemaphoreType.DMA,
            pltpu.SemaphoreType.DMA,
            pltpu.SemaphoreType.DMA,
            pltpu.SemaphoreType.DMA,
            pltpu.SemaphoreType.DMA,
            pltpu.SemaphoreType.DMA,
        ],
        compiler_params=pltpu.CompilerParams(needs_layout_passes=False),
    )
    def k(ei, p_t, ad_t, out, acc, wl_s, wl_d, st_s0, st_d0, st_s1, st_d1,
          rows0, rows1, ad_sl, lidx0, lidx1, wsc, zbuf,
          sem_a, sem_b, gsem0, gsem1, ssem0, ssem1):
        c = lax.axis_index("c")
        s = lax.axis_index("s")
        zvec = jnp.zeros((16,), jnp.float32)

        @pl.loop(0, ZROWS)
        def _(r):
            for kk in range(PCOLS // 16):
                zbuf[r, pl.ds(kk * 16, 16)] = zvec

        iota = lax.iota(jnp.int32, 16)

        def make_edge_body(rows, lidx):
            def one_edge(e, wsc_e):
                loc_b = plsc.load_gather(lidx, [iota * 0 + e])
                if mode == 1:
                    adidx = loc_b * 8 + (iota & 7)
                else:
                    adidx = loc_b
                ad = plsc.load_gather(ad_sl, [adidx])
                asad = rows[e, pl.ds(64, 16)]
                t = asad + ad
                t = jnp.maximum(t, 0.2 * t)
                w = jnp.exp(t)
                if mode == 1:
                    wsc[pl.ds(wsc_e * 16, 16)] = w
                    for kk in range(4):
                        bidx = (lax.shift_right_logical(iota, 3)
                                + (2 * kk + 16 * wsc_e))
                        wb = plsc.load_gather(wsc, [bidx])
                        rows[e, pl.ds(kk * 16, 16)] = (
                            wb * rows[e, pl.ds(kk * 16, 16)])
                    rows[e, pl.ds(64, 16)] = w
                else:
                    for kk in range(4):
                        rows[e, pl.ds(kk * 16, 16)] = (
                            w * rows[e, pl.ds(kk * 16, 16)])
                    rows[e, pl.ds(64, 16)] = w

            def edge_body(i, _):
                one_edge(2 * i, 0)
                one_edge(2 * i + 1, 1)
                return 0
            return edge_body

        def make_drain(lo):
            def compute_blk(g, rows, lidx):
                def lidx_body(i, _):
                    d16 = wl_d[pl.ds(g * GB + i * 16, 16)]
                    lidx[pl.ds(i * 16, 16)] = d16 - lo
                    return 0

                lax.fori_loop(0, GB // 16, lidx_body, 0)
                lax.fori_loop(0, GB // 2, make_edge_body(rows, lidx), 0)

            def drain(ptr):
                for kk in range(16):
                    wl_s[pl.ds(ptr + 16 * kk, 16)] = iota * 0
                    wl_d[pl.ds(ptr + 16 * kk, 16)] = iota * 0 + (lo + R)
                nb2 = lax.shift_right_logical(ptr + (2 * GB - 1), 8)

                def pair(ii, _):
                    g0 = 2 * ii
                    g1 = 2 * ii + 1
                    cg0 = pltpu.async_copy(
                        p_t.at[wl_s.at[pl.ds(g0 * GB, GB)]], rows0, gsem0)
                    cg1 = pltpu.async_copy(
                        p_t.at[wl_s.at[pl.ds(g1 * GB, GB)]], rows1, gsem1)
                    cg0.wait()
                    compute_blk(g0, rows0, lidx0)
                    cs0 = pltpu.async_copy(rows0, acc.at[lidx0], ssem0,
                                           add=True)
                    cg1.wait()
                    compute_blk(g1, rows1, lidx1)
                    cs1 = pltpu.async_copy(rows1, acc.at[lidx1], ssem1,
                                           add=True)
                    cs0.wait()
                    cs1.wait()
                    return 0

                lax.fori_loop(0, nb2, pair, 0)
                return jnp.int32(0)
            return drain

        npass_c = jnp.where(c == 0, (NPASS + 1) // 2, NPASS // 2)

        def pass_body(j, _):
            p = c + 2 * j
            if True:
                lo = p * R
                hi = lo + R
                drain = make_drain(lo)

                pltpu.sync_copy(ad_t.at[pl.ds(lo * adw, R * adw)], ad_sl)
                for z in range(6):
                    pltpu.sync_copy(
                        zbuf, acc.at[pl.ds(s * (6 * ZROWS) + z * ZROWS, ZROWS)])
                plsc.subcore_barrier()

                def fire(b, ss, sd, sem):
                    base_e = s * EPW + b * SB
                    pltpu.async_copy(ei.at[pl.ds(base_e, SB)], ss, sem)
                    pltpu.async_copy(ei.at[pl.ds(E + base_e, SB)], sd, sem)

                def wait_scan(ss, sd, sem):
                    pltpu.make_async_copy(ei.at[pl.ds(0, SB)], ss, sem).wait()
                    pltpu.make_async_copy(ei.at[pl.ds(0, SB)], sd, sem).wait()

                def process(ss, sd, ptr):
                    def lane(i, ptr):
                        d16 = sd[pl.ds(i * 16, 16)]
                        m = lax.shift_right_logical(d16, 12) == p
                        plsc.store_compressed(wl_d.at[pl.ds(ptr, 16)], d16,
                                              mask=m)
                        s16 = ss[pl.ds(i * 16, 16)]
                        plsc.store_compressed(wl_s.at[pl.ds(ptr, 16)], s16,
                                              mask=m)
                        cnt = jnp.max(plsc.all_reduce_population_count(m))
                        return ptr + cnt

                    return lax.fori_loop(0, SB // 16, lane, ptr)

                fire(0, st_s0, st_d0, sem_a)

                def scan_pair(i, ptr):
                    fire(2 * i + 1, st_s1, st_d1, sem_b)
                    wait_scan(st_s0, st_d0, sem_a)
                    ptr = process(st_s0, st_d0, ptr)

                    @pl.when(i < NSB // 2 - 1)
                    def _():
                        fire(2 * i + 2, st_s0, st_d0, sem_a)

                    wait_scan(st_s1, st_d1, sem_b)
                    ptr = process(st_s1, st_d1, ptr)
                    ptr = lax.cond(ptr >= THR, drain, lambda q: q, ptr)
                    return ptr

                ptr = lax.fori_loop(0, NSB // 2, scan_pair, jnp.int32(0))
                drain(ptr)
                plsc.subcore_barrier()
                pltpu.sync_copy(acc.at[pl.ds(s * (R // NS), R // NS)],
                                out.at[pl.ds(lo + s * (R // NS), R // NS)])
                plsc.subcore_barrier()
            return 0

        lax.fori_loop(0, npass_c, pass_body, 0)

    return k


@functools.lru_cache(maxsize=2)
def _sc_layer(mode):
    return _sc_msgpass(mode)


def _sc_layer1(ei, p_t, ad_2d):
    ad = jnp.pad(ad_2d, ((0, TPAD - NPAD), (0, 0))).reshape(TPAD * 8)
    return _sc_layer(1)(ei.reshape(2 * E), p_t, ad)


def _sc_layer2(ei, p_t, ad_2d):
    ad = jnp.pad(ad_2d, ((0, TPAD - NPAD), (0, 0))).reshape(TPAD)
    return _sc_layer(2)(ei.reshape(2 * E), p_t, ad)


def _k1_body(x_ref, w1_ref, a1_ref, p_ref, ad_ref):
    h = jnp.dot(x_ref[...], w1_ref[...], preferred_element_type=jnp.float32)
    t = jnp.dot(h, a1_ref[...], preferred_element_type=jnp.float32)
    zp = jnp.zeros((BN, PCOLS - 72), jnp.float32)
    p_ref[...] = jnp.concatenate([h, t[:, :8], zp], axis=1)
    ad_ref[...] = t[:, 8:]


def _k3_body(acc_ref, b1_ref, rep_ref, wfc1_ref, bfc1_ref, wfc2_ref, bfc2_ref,
             u_ref, st_ref):
    i = pl.program_id(0)
    acc = acc_ref[...]
    den = jnp.dot(acc[:, 64:72], rep_ref[...],
                  preferred_element_type=jnp.float32)
    h = jax.nn.relu(acc[:, :64] / (den + 1e-16) + b1_ref[...])
    t = jax.nn.relu(jnp.dot(h, wfc1_ref[...],
                            preferred_element_type=jnp.float32) + bfc1_ref[...])
    u = jax.nn.relu(jnp.dot(t, wfc2_ref[...],
                            preferred_element_type=jnp.float32)
                    + bfc2_ref[...] + h)
    u_ref[...] = u
    rid = lax.broadcasted_iota(jnp.int32, (BN, 1), 0) + i * BN
    um = jnp.where(rid < N, u, 0.0)

    @pl.when(i == 0)
    def _():
        st_ref[...] = jnp.zeros_like(st_ref)

    st_ref[...] += jnp.concatenate(
        [jnp.sum(um).reshape(1, 1), jnp.sum(um * um).reshape(1, 1)], axis=1)


def _k4_body(u_ref, st_ref, g1_ref, be1_ref, w2_ref, a2_ref,
             gn_ref, p_ref, ad_ref):
    mean, std = _stats(st_ref)
    gn = (u_ref[...] - mean) / (std + EPS) * g1_ref[...] + be1_ref[...]
    gn_ref[...] = gn
    h2 = jnp.dot(gn, w2_ref[...], preferred_element_type=jnp.float32)
    t2 = jnp.dot(h2, a2_ref[...], preferred_element_type=jnp.float32)
    zp = jnp.zeros((BN, PCOLS - 80), jnp.float32)
    as2 = jnp.broadcast_to(t2[:, 0:1], (BN, 16))
    p_ref[...] = jnp.concatenate([h2, as2, zp], axis=1)
    ad_ref[...] = t2[:, 1:2]


def _k5_body(acc_ref, b2_ref, gn_ref, wfc3_ref, bfc3_ref, wfc4_ref, bfc4_ref,
             v_ref, st_ref):
    i = pl.program_id(0)
    acc = acc_ref[...]
    h = acc[:, :64] / (acc[:, 64:65] + 1e-16) + b2_ref[...]
    t = jax.nn.relu(jnp.dot(h, wfc3_ref[...],
                            preferred_element_type=jnp.float32) + bfc3_ref[...])
    v = (jnp.dot(t, wfc4_ref[...], preferred_element_type=jnp.float32)
         + bfc4_ref[...] + gn_ref[...])
    v_ref[...] = v
    rid = lax.broadcasted_iota(jnp.int32, (BN, 1), 0) + i * BN
    vm = jnp.where(rid < N, v, 0.0)

    @pl.when(i == 0)
    def _():
        st_ref[...] = jnp.zeros_like(st_ref)

    st_ref[...] += jnp.concatenate(
        [jnp.sum(vm).reshape(1, 1), jnp.sum(vm * vm).reshape(1, 1)], axis=1)


def _stats(st_ref):
    st = st_ref[...]
    mean = st[0, 0] / (N * D)
    var = st[0, 1] / (N * D) - mean * mean
    std = jnp.sqrt(jnp.maximum(var, 0.0))
    return mean, std


def _k6_body(v_ref, st_ref, g2_ref, be2_ref, wf_ref, bf_ref, b_ref, out_ref):
    i = pl.program_id(0)
    mean, std = _stats(st_ref)
    w = (v_ref[...] - mean) / (std + EPS) * g2_ref[...] + be2_ref[...]
    y = jnp.dot(w, wf_ref[...], preferred_element_type=jnp.float32) + bf_ref[...]
    bvals = b_ref[...].reshape(1, BN)
    gids = lax.broadcasted_iota(jnp.int32, (G, 1), 0).astype(jnp.float32)
    oh = (bvals == gids).astype(jnp.float32)
    ones = jnp.ones((BN, 1), jnp.float32)
    cy = jnp.dot(oh, y, preferred_element_type=jnp.float32)
    cc = jnp.dot(oh, ones, preferred_element_type=jnp.float32)

    @pl.when(i == 0)
    def _():
        out_ref[...] = jnp.zeros_like(out_ref)

    out_ref[...] += jnp.concatenate([cy, cc], axis=1)


def _k7_body(p_ref, o_ref):
    o_ref[...] = p_ref[:, 0:1] / jnp.clip(p_ref[:, 1:2], 1.0, None)


def _full(shape):
    return pl.BlockSpec(shape, lambda i: tuple(0 for _ in shape))


def kernel(x, edge_index, batch, W1, a_src1, a_dst1, b1, Wfc1, bfc1, Wfc2,
           bfc2, g1, be1, W2, a_src2, a_dst2, b2, Wfc3, bfc3, Wfc4, bfc4,
           g2, be2, Wf, bf):
    f32 = jnp.float32
    as1m = a_src1.reshape(HEADS, HID)
    ad1m = a_dst1.reshape(HEADS, HID)
    eye = jnp.eye(HEADS, dtype=f32)
    A_s = (eye[:, None, :] * as1m[:, :, None]).reshape(HEADS * HID, HEADS)
    A_d = (eye[:, None, :] * ad1m[:, :, None]).reshape(HEADS * HID, HEADS)
    A1 = jnp.concatenate([A_s, A_d], axis=1)
    A2 = jnp.concatenate([a_src2.reshape(D, 1), a_dst2.reshape(D, 1)], axis=1)
    Rep = jnp.broadcast_to(eye[:, :, None], (8, 8, 8)).reshape(8, 64)
    xp = jnp.pad(x, ((0, NPAD - N), (0, 0)))
    batchf = jnp.pad(batch, (0, NPAD - N), constant_values=G).astype(f32)
    batchf = batchf.reshape(NB, 1, BN)
    b1r, b2r = b1.reshape(1, D), b2.reshape(1, D)
    bfc1r, bfc2r = bfc1.reshape(1, D), bfc2.reshape(1, D)
    bfc3r, bfc4r = bfc3.reshape(1, D), bfc4.reshape(1, D)
    g1r, be1r = g1.reshape(1, D), be1.reshape(1, D)
    g2r, be2r = g2.reshape(1, D), be2.reshape(1, D)
    bfr = bf.reshape(1, 1)

    def rs(w):
        return pl.BlockSpec((BN, w), lambda i: (i, 0))

    P1, AD1 = pl.pallas_call(
        _k1_body,
        grid=(NB,),
        in_specs=[rs(D_IN), _full((D_IN, D)), _full((D, 16))],
        out_specs=[rs(PCOLS), rs(8)],
        out_shape=[jax.ShapeDtypeStruct((NPAD, PCOLS), f32),
                   jax.ShapeDtypeStruct((NPAD, 8), f32)],
    )(xp, W1, A1)

    ACC1 = _sc_layer1(edge_index, P1, AD1)

    u, st1 = pl.pallas_call(
        _k3_body,
        grid=(NB,),
        in_specs=[rs(PCOLS), _full((1, D)), _full((8, D)), _full((D, D)),
                  _full((1, D)), _full((D, D)), _full((1, D))],
        out_specs=[rs(D), _full((1, 2))],
        out_shape=[jax.ShapeDtypeStruct((NPAD, D), f32),
                   jax.ShapeDtypeStruct((1, 2), f32)],
    )(ACC1, b1r, Rep, Wfc1, bfc1r, Wfc2, bfc2r)

    gn, P2, AD2 = pl.pallas_call(
        _k4_body,
        grid=(NB,),
        in_specs=[rs(D), _full((1, 2)), _full((1, D)), _full((1, D)),
                  _full((D, D)), _full((D, 2))],
        out_specs=[rs(D), rs(PCOLS), rs(1)],
        out_shape=[jax.ShapeDtypeStruct((NPAD, D), f32),
                   jax.ShapeDtypeStruct((NPAD, PCOLS), f32),
                   jax.ShapeDtypeStruct((NPAD, 1), f32)],
    )(u, st1, g1r, be1r, W2, A2)

    ACC2 = _sc_layer2(edge_index, P2, AD2)

    v, st2 = pl.pallas_call(
        _k5_body,
        grid=(NB,),
        in_specs=[rs(PCOLS), _full((1, D)), rs(D), _full((D, D)),
                  _full((1, D)), _full((D, D)), _full((1, D))],
        out_specs=[rs(D), _full((1, 2))],
        out_shape=[jax.ShapeDtypeStruct((NPAD, D), f32),
                   jax.ShapeDtypeStruct((1, 2), f32)],
    )(ACC2, b2r, gn, Wfc3, bfc3r, Wfc4, bfc4r)

    pool = pl.pallas_call(
        _k6_body,
        grid=(NB,),
        in_specs=[rs(D), _full((1, 2)), _full((1, D)), _full((1, D)),
                  _full((D, 1)), _full((1, 1)),
                  pl.BlockSpec((1, 1, BN), lambda i: (i, 0, 0))],
        out_specs=_full((G, 2)),
        out_shape=jax.ShapeDtypeStruct((G, 2), f32),
    )(v, st2, g2r, be2r, Wf, bfr, batchf)

    out = pl.pallas_call(
        _k7_body,
        out_shape=jax.ShapeDtypeStruct((G, 1), f32),
    )(pool)
    return out
